# baseline (speedup 1.0000x reference)
import functools

import jax
import jax.numpy as jnp
from jax import lax
from jax.experimental import pallas as pl
from jax.experimental.pallas import tpu as pltpu
from jax.experimental.pallas import tpu_sc as plsc

B = 4
S = 4096
D = 768
LANES = 16
NUM_CORES = 2
NUM_SUBCORES = 16
NW = NUM_CORES * NUM_SUBCORES
SPW = S // NW
C = 32
NSC = SPW // C
STEPS = NSC * B
VECS_PER_ROW = D // LANES


def _body(x_hbm, table_hbm, pos_hbm, out_hbm, idx_v, pos_v, rows_v,
          g0, g1, o0, o1, p0, p1):
    wid = lax.axis_index("s") * NUM_CORES + lax.axis_index("c")
    s0 = wid * SPW
    gsem = [g0, g1]
    osem = [o0, o1]
    psem = [p0, p1]

    for b in range(B):
        pltpu.sync_copy(x_hbm.at[pl.ds(b * S + s0, SPW)], idx_v.at[b])

    gdesc = [None, None]
    odesc = [None, None]
    pdesc = [None, None]

    pdesc[0] = pltpu.async_copy(pos_hbm.at[pl.ds(s0, C)], pos_v.at[0], psem[0])
    gdesc[0] = pltpu.async_copy(
        table_hbm.at[idx_v.at[0, pl.ds(0, C)]], rows_v.at[0], gsem[0])

    for t in range(STEPS):
        sc, b = divmod(t, B)
        cur = t % 2
        if t + 1 < STEPS:
            sc1, b1 = divmod(t + 1, B)
            nxt = (t + 1) % 2
            if odesc[nxt] is not None:
                odesc[nxt].wait()
            gdesc[nxt] = pltpu.async_copy(
                table_hbm.at[idx_v.at[b1, pl.ds(sc1 * C, C)]],
                rows_v.at[nxt], gsem[nxt])
        if b == 0 and sc + 1 < NSC:
            pslot = (sc + 1) % 2
            pdesc[pslot] = pltpu.async_copy(
                pos_hbm.at[pl.ds(s0 + (sc + 1) * C, C)], pos_v.at[pslot],
                psem[pslot])
        if b == 0:
            pdesc[sc % 2].wait()
        gdesc[cur].wait()

        def add_row(r, carry, cur=cur, ps=sc % 2):
            for j in range(VECS_PER_ROW):
                sl = pl.ds(j * LANES, LANES)
                rows_v[cur, r, sl] = rows_v[cur, r, sl] + pos_v[ps, r, sl]
            return carry

        lax.fori_loop(0, C, add_row, 0)
        odesc[cur] = pltpu.async_copy(
            rows_v.at[cur], out_hbm.at[pl.ds(b * S + s0 + sc * C, C)],
            osem[cur])

    odesc[(STEPS - 1) % 2].wait()
    odesc[(STEPS - 2) % 2].wait()


@jax.jit
def _embed(x_flat, table, pos_table):
    mesh = plsc.VectorSubcoreMesh(core_axis_name="c", subcore_axis_name="s")
    kfn = functools.partial(
        pl.kernel,
        out_type=jax.ShapeDtypeStruct((B * S, D), jnp.float32),
        mesh=mesh,
        scratch_types=[
            pltpu.VMEM((B, SPW), jnp.int32),
            pltpu.VMEM((2, C, D), jnp.float32),
            pltpu.VMEM((2, C, D), jnp.float32),
            pltpu.SemaphoreType.DMA,
            pltpu.SemaphoreType.DMA,
            pltpu.SemaphoreType.DMA,
            pltpu.SemaphoreType.DMA,
            pltpu.SemaphoreType.DMA,
            pltpu.SemaphoreType.DMA,
        ],
    )(_body)
    return kfn(x_flat, table, pos_table)


def kernel(x, table, pos_table):
    x_flat = x.reshape(B * S).astype(jnp.int32)
    out = _embed(x_flat, table, pos_table)
    return out.reshape(B, S, D)
